# baseline (speedup 1.0000x reference)
import functools

import jax
import jax.numpy as jnp
from jax.experimental import pallas as pl
from jax.experimental.pallas import tpu as pltpu

_EPS = 1e-5

_D = {0: (0, -1), 1: (1, 0)}
_PHASES = ((0, 0), (0, 1), (1, 0), (1, 1))


def _qshift_h(t, d):
    z = jnp.zeros_like(t[..., 0:4, :])
    if d == 1:
        return jnp.concatenate([t[..., 4:, :], z], axis=-2)
    return jnp.concatenate([z, t[..., :12, :]], axis=-2)


def _qshift_w(t, d):
    z = jnp.zeros_like(t[..., 0:1, :])
    if d == 1:
        s = jnp.concatenate([t[..., 1:, :], z], axis=-2)
        bad = 3
    else:
        s = jnp.concatenate([z, t[..., :-1, :]], axis=-2)
        bad = 0
    q = jax.lax.broadcasted_iota(jnp.int32, (16, 1), 0) % 4
    return jnp.where(q == bad, jnp.zeros_like(s), s)


def _tap_h(y, d):
    if d == 0:
        return y
    p = y.shape[1]
    if d == 1:
        wrap = _qshift_h(y[:, 0:1], 1)
        if p == 1:
            return wrap
        return jnp.concatenate([y[:, 1:], wrap], axis=1)
    wrap = _qshift_h(y[:, p - 1:p], -1)
    if p == 1:
        return wrap
    return jnp.concatenate([wrap, y[:, :p - 1]], axis=1)


def _tap_w(y, d):
    if d == 0:
        return y
    p = y.shape[2]
    if d == 1:
        wrap = _qshift_w(y[:, :, 0:1], 1)
        if p == 1:
            return wrap
        return jnp.concatenate([y[:, :, 1:], wrap], axis=2)
    wrap = _qshift_w(y[:, :, p - 1:p], -1)
    if p == 1:
        return wrap
    return jnp.concatenate([wrap, y[:, :, :p - 1]], axis=2)


def _taps9(y):
    bsz, p, _, s, c = y.shape
    zt = {}
    for dh in (-1, 0, 1):
        th = _tap_h(y, dh)
        for dw in (-1, 0, 1):
            zt[(dh, dw)] = _tap_w(th, dw).reshape(bsz * p * p * s, c)
    return zt


def _deconv_phases(y, w_ref, b_ref):
    zt = _taps9(y)
    outs = []
    for ph, (a, b) in enumerate(_PHASES):
        xt = jnp.concatenate(
            [zt[(dh, dw)] for dh in _D[a] for dw in _D[b]], axis=1)
        outs.append(jnp.dot(xt, w_ref[ph],
                            preferred_element_type=jnp.float32) + b_ref[...])
    return outs


def _k1(x_ref, w_ref, b_ref, o_ref, s_ref, q_ref, *, bsz):
    y = x_ref[...].reshape(bsz, 1, 1, 16, 512)
    outs = _deconv_phases(y, w_ref, b_ref)
    ssum = jnp.zeros((256,), jnp.float32)
    ssq = jnp.zeros((256,), jnp.float32)
    for ph, (a, b) in enumerate(_PHASES):
        o = outs[ph]
        ssum = ssum + jnp.sum(o, axis=0)
        ssq = ssq + jnp.sum(o * o, axis=0)
        o_ref[:, a, b] = o.astype(jnp.bfloat16).reshape(bsz, 16, 256)
    s_ref[...] = ssum.reshape(1, 1, 256)
    q_ref[...] = ssq.reshape(1, 1, 256)


def _k2(x_ref, sc_ref, sf_ref, w_ref, b_ref, o_ref, s_ref, q_ref, *, bsz):
    x = x_ref[...].astype(jnp.float32)
    x = jnp.maximum(x * sc_ref[...] + sf_ref[...], 0.0).astype(jnp.bfloat16)
    outs = _deconv_phases(x, w_ref, b_ref)
    ssum = jnp.zeros((128,), jnp.float32)
    ssq = jnp.zeros((128,), jnp.float32)
    for ph, (a, b) in enumerate(_PHASES):
        o = outs[ph]
        ssum = ssum + jnp.sum(o, axis=0)
        ssq = ssq + jnp.sum(o * o, axis=0)
        o_ref[:, :, a, :, b] = o.astype(jnp.bfloat16).reshape(
            bsz, 2, 2, 16, 128)
    s_ref[...] = ssum.reshape(1, 1, 128)
    q_ref[...] = ssq.reshape(1, 1, 128)


def _k34(x_ref, sc_ref, sf_ref, w3_ref, b3_ref, w4_ref, b4_ref, o_ref, *,
         bsz):
    x = x_ref[...].astype(jnp.float32)
    x = jnp.maximum(x * sc_ref[...] + sf_ref[...], 0.0).astype(jnp.bfloat16)

    zt = _taps9(x)
    xt = jnp.concatenate(
        [zt[(dh, dw)] for dh in (-1, 0, 1) for dw in (-1, 0, 1)], axis=1)
    r = jnp.dot(xt, w3_ref[...],
                preferred_element_type=jnp.float32) + b3_ref[...]
    r = jnp.maximum(r, 0.0).astype(jnp.bfloat16)

    r6 = r.reshape(bsz, 4, 4, 16, 2, 2, 64)
    y3f = r6.transpose(0, 1, 4, 3, 2, 5, 6).reshape(bsz, 8, 16, 512)

    cols = []
    for dh in (-1, 0, 1):
        t = _tap_h(y3f, dh)
        left = _qshift_w(t[..., 7 * 64:], -1)
        right = _qshift_w(t[..., :64], 1)
        cols.append(jnp.concatenate([left, t, right], axis=-1))
    xtb = jnp.concatenate(cols, axis=-1).reshape(bsz * 128, 1920)
    y4 = jnp.dot(xtb, w4_ref[...],
                 preferred_element_type=jnp.float32) + b4_ref[...]
    o_ref[...] = y4.astype(jnp.bfloat16).reshape(bsz, 8, 16, 96)


def _mega(x_ref, w1_ref, b1_ref, g0_ref, be0_ref, w2_ref, b2_ref,
          g1_ref, be1_ref, w3_ref, b3_ref, w4_ref, b4_ref, o_ref,
          y1_scr, y2_scr, st1_scr, st2_scr, *, n):
    i = pl.program_id(0)

    @pl.when(i == 0)
    def _init():
        st1_scr[...] = jnp.zeros((2, 256), jnp.float32)
        st2_scr[...] = jnp.zeros((2, 128), jnp.float32)

    @pl.when(i < 8)
    def _stage_a():
        bsz = 32
        y = x_ref[...].reshape(bsz, 1, 1, 16, 512)
        outs = _deconv_phases(y, w1_ref, b1_ref)
        ssum = jnp.zeros((1, 256), jnp.float32)
        ssq = jnp.zeros((1, 256), jnp.float32)
        for ph, (a, b) in enumerate(_PHASES):
            o = outs[ph]
            ssum = ssum + jnp.sum(o, axis=0).reshape(1, 256)
            ssq = ssq + jnp.sum(o * o, axis=0).reshape(1, 256)
            y1_scr[pl.ds(i * bsz, bsz), a, b] = o.astype(
                jnp.bfloat16).reshape(bsz, 16, 256)
        st1_scr[0:1] = st1_scr[0:1] + ssum
        st1_scr[1:2] = st1_scr[1:2] + ssq

    @pl.when(jnp.logical_and(i >= 8, i < 24))
    def _stage_b():
        bsz = 16
        j = i - 8
        inv = 1.0 / float(n * 64)
        mean = st1_scr[0:1] * inv
        var = st1_scr[1:2] * inv - mean * mean
        sc = g0_ref[...] * jax.lax.rsqrt(var + _EPS)
        sf = be0_ref[...] - mean * sc
        x = y1_scr[pl.ds(j * bsz, bsz)].astype(jnp.float32)
        x = jnp.maximum(x * sc + sf, 0.0).astype(jnp.bfloat16)
        outs = _deconv_phases(x, w2_ref, b2_ref)
        ssum = jnp.zeros((1, 128), jnp.float32)
        ssq = jnp.zeros((1, 128), jnp.float32)
        for ph, (a, b) in enumerate(_PHASES):
            o = outs[ph]
            ssum = ssum + jnp.sum(o, axis=0).reshape(1, 128)
            ssq = ssq + jnp.sum(o * o, axis=0).reshape(1, 128)
            y2_scr[pl.ds(j * bsz, bsz), :, a, :, b] = o.astype(
                jnp.bfloat16).reshape(bsz, 2, 2, 16, 128)
        st2_scr[0:1] = st2_scr[0:1] + ssum
        st2_scr[1:2] = st2_scr[1:2] + ssq

    @pl.when(i >= 24)
    def _stage_c():
        bsz = 8
        k = i - 24
        inv = 1.0 / float(n * 256)
        mean = st2_scr[0:1] * inv
        var = st2_scr[1:2] * inv - mean * mean
        sc = g1_ref[...] * jax.lax.rsqrt(var + _EPS)
        sf = be1_ref[...] - mean * sc
        x = y2_scr[pl.ds(k * bsz, bsz)].reshape(
            bsz, 4, 4, 16, 128).astype(jnp.float32)
        x = jnp.maximum(x * sc + sf, 0.0).astype(jnp.bfloat16)

        zt = _taps9(x)
        xt = jnp.concatenate(
            [zt[(dh, dw)] for dh in (-1, 0, 1) for dw in (-1, 0, 1)],
            axis=1)
        r = jnp.dot(xt, w3_ref[...],
                    preferred_element_type=jnp.float32) + b3_ref[...]
        r = jnp.maximum(r, 0.0).astype(jnp.bfloat16)
        r6 = r.reshape(bsz, 4, 4, 16, 2, 2, 64)
        y3f = r6.transpose(0, 1, 4, 3, 2, 5, 6).reshape(bsz, 8, 16, 512)
        cols = []
        for dh in (-1, 0, 1):
            t = _tap_h(y3f, dh)
            left = _qshift_w(t[..., 7 * 64:], -1)
            right = _qshift_w(t[..., :64], 1)
            cols.append(jnp.concatenate([left, t, right], axis=-1))
        xtb = jnp.concatenate(cols, axis=-1).reshape(bsz * 128, 1920)
        y4 = jnp.dot(xtb, w4_ref[...],
                     preferred_element_type=jnp.float32) + b4_ref[...]
        o_ref[...] = y4.astype(jnp.bfloat16).reshape(bsz, 8, 16, 96)


def _bn_affine(s, q, gamma, beta, count):
    inv = 1.0 / float(count)
    mean = jnp.sum(s, axis=0).reshape(1, -1) * inv
    var = jnp.sum(q, axis=0).reshape(1, -1) * inv - mean * mean
    scale = gamma * jax.lax.rsqrt(var + _EPS)
    shift = beta - mean * scale
    return scale, shift


def _build_9tap(wph):
    cin = wph.shape[1] // 4
    cout = wph.shape[2]
    w9 = jnp.zeros((9, cin, 4 * cout), wph.dtype)
    for ph, (a, b) in enumerate(_PHASES):
        for j in range(4):
            dh = _D[a][j // 2]
            dw = _D[b][j % 2]
            k = (dh + 1) * 3 + (dw + 1)
            w9 = w9.at[k, :, ph * cout:(ph + 1) * cout].set(
                wph[ph, j * cin:(j + 1) * cin, :])
    return w9.reshape(9 * cin, 4 * cout).astype(jnp.bfloat16)


def _build_l4_banded(wph3, b3):
    w4 = jnp.zeros((3, 10, 64, 2, 16, 3), wph3.dtype)
    for ah in (0, 1):
        for aw in (0, 1):
            ph = ah * 2 + aw
            for jh, dh in enumerate(_D[ah]):
                for jw, dw in enumerate(_D[aw]):
                    j = jh * 2 + jw
                    blk = wph3[ph, j * 64:(j + 1) * 64, :]
                    for psi in range(8):
                        if dw == -1 and psi == 0:
                            psa = 0
                        elif dw == 1 and psi == 7:
                            psa = 9
                        else:
                            psa = psi + dw + 1
                        w4 = w4.at[dh + 1, psa, :, ah,
                                   2 * psi + aw, :].set(blk)
    b96 = jnp.tile(b3, (1, 32))
    return w4.reshape(1920, 96).astype(jnp.bfloat16), b96


def kernel(wph0, b0, wph1, b1, wph2, b2, wph3, b3,
           gamma0, beta0, gamma1, beta1, x):
    n = x.shape[0]
    xh = jnp.transpose(x, (0, 2, 3, 1)).reshape(n, 16, 512).astype(
        jnp.bfloat16)
    w1 = wph0.astype(jnp.bfloat16)
    w2 = wph1.astype(jnp.bfloat16)
    w3 = _build_9tap(wph2)
    b3t = jnp.tile(b2, (1, 4))
    w4, b96 = _build_l4_banded(wph3, b3)

    o = pl.pallas_call(
        functools.partial(_mega, n=n),
        out_shape=jax.ShapeDtypeStruct((n, 8, 16, 96), jnp.bfloat16),
        grid=(56,),
        in_specs=[
            pl.BlockSpec((32, 16, 512),
                         lambda i: (jnp.minimum(i, 7), 0, 0)),
            pl.BlockSpec((4, 2048, 256), lambda i: (0, 0, 0)),
            pl.BlockSpec((1, 256), lambda i: (0, 0)),
            pl.BlockSpec((1, 256), lambda i: (0, 0)),
            pl.BlockSpec((1, 256), lambda i: (0, 0)),
            pl.BlockSpec((4, 1024, 128), lambda i: (0, 0, 0)),
            pl.BlockSpec((1, 128), lambda i: (0, 0)),
            pl.BlockSpec((1, 128), lambda i: (0, 0)),
            pl.BlockSpec((1, 128), lambda i: (0, 0)),
            pl.BlockSpec((1152, 256), lambda i: (0, 0)),
            pl.BlockSpec((1, 256), lambda i: (0, 0)),
            pl.BlockSpec((1920, 96), lambda i: (0, 0)),
            pl.BlockSpec((1, 96), lambda i: (0, 0)),
        ],
        out_specs=pl.BlockSpec((8, 8, 16, 96),
                               lambda i: (jnp.maximum(i - 24, 0), 0, 0, 0)),
        scratch_shapes=[
            pltpu.VMEM((n, 2, 2, 16, 256), jnp.bfloat16),
            pltpu.VMEM((n, 2, 2, 2, 2, 16, 128), jnp.bfloat16),
            pltpu.VMEM((2, 256), jnp.float32),
            pltpu.VMEM((2, 128), jnp.float32),
        ],
        compiler_params=pltpu.CompilerParams(
            dimension_semantics=("arbitrary",),
            vmem_limit_bytes=60 * 1024 * 1024),
    )(xh, w1, b0, gamma0, beta0, w2, b1, gamma1, beta1, w3, b3t, w4, b96)

    r = o.reshape(n, 8, 4, 4, 2, 16, 3)
    return r.transpose(0, 6, 2, 1, 4, 3, 5).reshape(
        n, 3, 64, 64).astype(jnp.float32)


# speedup vs baseline: 1.0385x; 1.0385x over previous
import functools

import jax
import jax.numpy as jnp
from jax.experimental import pallas as pl
from jax.experimental.pallas import tpu as pltpu

_EPS = 1e-5

_D = {0: (0, -1), 1: (1, 0)}
_PHASES = ((0, 0), (0, 1), (1, 0), (1, 1))


def _qshift_h(t, d):
    z = jnp.zeros_like(t[..., 0:4, :])
    if d == 1:
        return jnp.concatenate([t[..., 4:, :], z], axis=-2)
    return jnp.concatenate([z, t[..., :12, :]], axis=-2)


def _qshift_w(t, d):
    z = jnp.zeros_like(t[..., 0:1, :])
    if d == 1:
        s = jnp.concatenate([t[..., 1:, :], z], axis=-2)
        bad = 3
    else:
        s = jnp.concatenate([z, t[..., :-1, :]], axis=-2)
        bad = 0
    q = jax.lax.broadcasted_iota(jnp.int32, (16, 1), 0) % 4
    return jnp.where(q == bad, jnp.zeros_like(s), s)


def _tap_h(y, d):
    if d == 0:
        return y
    p = y.shape[1]
    if d == 1:
        wrap = _qshift_h(y[:, 0:1], 1)
        if p == 1:
            return wrap
        return jnp.concatenate([y[:, 1:], wrap], axis=1)
    wrap = _qshift_h(y[:, p - 1:p], -1)
    if p == 1:
        return wrap
    return jnp.concatenate([wrap, y[:, :p - 1]], axis=1)


def _tap_w(y, d):
    if d == 0:
        return y
    p = y.shape[2]
    if d == 1:
        wrap = _qshift_w(y[:, :, 0:1], 1)
        if p == 1:
            return wrap
        return jnp.concatenate([y[:, :, 1:], wrap], axis=2)
    wrap = _qshift_w(y[:, :, p - 1:p], -1)
    if p == 1:
        return wrap
    return jnp.concatenate([wrap, y[:, :, :p - 1]], axis=2)


def _taps9(y):
    bsz, p, _, s, c = y.shape
    zt = {}
    for dh in (-1, 0, 1):
        th = _tap_h(y, dh)
        for dw in (-1, 0, 1):
            zt[(dh, dw)] = _tap_w(th, dw).reshape(bsz * p * p * s, c)
    return zt


def _deconv_phases(y, w_ref, b_ref):
    zt = _taps9(y)
    outs = []
    for ph, (a, b) in enumerate(_PHASES):
        xt = jnp.concatenate(
            [zt[(dh, dw)] for dh in _D[a] for dw in _D[b]], axis=1)
        outs.append(jnp.dot(xt, w_ref[ph],
                            preferred_element_type=jnp.float32) + b_ref[...])
    return outs


def _k1(x_ref, w_ref, b_ref, o_ref, s_ref, q_ref, *, bsz):
    y = x_ref[...].reshape(bsz, 1, 1, 16, 512)
    outs = _deconv_phases(y, w_ref, b_ref)
    ssum = jnp.zeros((256,), jnp.float32)
    ssq = jnp.zeros((256,), jnp.float32)
    for ph, (a, b) in enumerate(_PHASES):
        o = outs[ph]
        ssum = ssum + jnp.sum(o, axis=0)
        ssq = ssq + jnp.sum(o * o, axis=0)
        o_ref[:, a, b] = o.astype(jnp.bfloat16).reshape(bsz, 16, 256)
    s_ref[...] = ssum.reshape(1, 1, 256)
    q_ref[...] = ssq.reshape(1, 1, 256)


def _k2(x_ref, sc_ref, sf_ref, w_ref, b_ref, o_ref, s_ref, q_ref, *, bsz):
    x = x_ref[...].astype(jnp.float32)
    x = jnp.maximum(x * sc_ref[...] + sf_ref[...], 0.0).astype(jnp.bfloat16)
    outs = _deconv_phases(x, w_ref, b_ref)
    ssum = jnp.zeros((128,), jnp.float32)
    ssq = jnp.zeros((128,), jnp.float32)
    for ph, (a, b) in enumerate(_PHASES):
        o = outs[ph]
        ssum = ssum + jnp.sum(o, axis=0)
        ssq = ssq + jnp.sum(o * o, axis=0)
        o_ref[:, :, a, :, b] = o.astype(jnp.bfloat16).reshape(
            bsz, 2, 2, 16, 128)
    s_ref[...] = ssum.reshape(1, 1, 128)
    q_ref[...] = ssq.reshape(1, 1, 128)


def _k34(x_ref, sc_ref, sf_ref, w3_ref, b3_ref, w4_ref, b4_ref, o_ref, *,
         bsz):
    x = x_ref[...].astype(jnp.float32)
    x = jnp.maximum(x * sc_ref[...] + sf_ref[...], 0.0).astype(jnp.bfloat16)

    zt = _taps9(x)
    xt = jnp.concatenate(
        [zt[(dh, dw)] for dh in (-1, 0, 1) for dw in (-1, 0, 1)], axis=1)
    r = jnp.dot(xt, w3_ref[...],
                preferred_element_type=jnp.float32) + b3_ref[...]
    r = jnp.maximum(r, 0.0).astype(jnp.bfloat16)

    r6 = r.reshape(bsz, 4, 4, 16, 2, 2, 64)
    y3f = r6.transpose(0, 1, 4, 3, 2, 5, 6).reshape(bsz, 8, 16, 512)

    cols = []
    for dh in (-1, 0, 1):
        t = _tap_h(y3f, dh)
        left = _qshift_w(t[..., 7 * 64:], -1)
        right = _qshift_w(t[..., :64], 1)
        cols.append(jnp.concatenate([left, t, right], axis=-1))
    xtb = jnp.concatenate(cols, axis=-1).reshape(bsz * 128, 1920)
    y4 = jnp.dot(xtb, w4_ref[...],
                 preferred_element_type=jnp.float32) + b4_ref[...]
    o_ref[...] = y4.astype(jnp.bfloat16).reshape(bsz, 8, 16, 96)


def _bn_affine(s, q, gamma, beta, count):
    inv = 1.0 / float(count)
    mean = jnp.sum(s, axis=0).reshape(1, -1) * inv
    var = jnp.sum(q, axis=0).reshape(1, -1) * inv - mean * mean
    scale = gamma * jax.lax.rsqrt(var + _EPS)
    shift = beta - mean * scale
    return scale, shift


def _build_9tap(wph):
    cin = wph.shape[1] // 4
    cout = wph.shape[2]
    w9 = jnp.zeros((9, cin, 4 * cout), wph.dtype)
    for ph, (a, b) in enumerate(_PHASES):
        for j in range(4):
            dh = _D[a][j // 2]
            dw = _D[b][j % 2]
            k = (dh + 1) * 3 + (dw + 1)
            w9 = w9.at[k, :, ph * cout:(ph + 1) * cout].set(
                wph[ph, j * cin:(j + 1) * cin, :])
    return w9.reshape(9 * cin, 4 * cout).astype(jnp.bfloat16)


def _build_l4_banded(wph3, b3):
    w4 = jnp.zeros((3, 10, 64, 2, 16, 3), wph3.dtype)
    for ah in (0, 1):
        for aw in (0, 1):
            ph = ah * 2 + aw
            for jh, dh in enumerate(_D[ah]):
                for jw, dw in enumerate(_D[aw]):
                    j = jh * 2 + jw
                    blk = wph3[ph, j * 64:(j + 1) * 64, :]
                    for psi in range(8):
                        if dw == -1 and psi == 0:
                            psa = 0
                        elif dw == 1 and psi == 7:
                            psa = 9
                        else:
                            psa = psi + dw + 1
                        w4 = w4.at[dh + 1, psa, :, ah,
                                   2 * psi + aw, :].set(blk)
    b96 = jnp.tile(b3, (1, 32))
    return w4.reshape(1920, 96).astype(jnp.bfloat16), b96


def kernel(wph0, b0, wph1, b1, wph2, b2, wph3, b3,
           gamma0, beta0, gamma1, beta1, x):
    n = x.shape[0]
    xh = jnp.transpose(x, (0, 2, 3, 1)).reshape(n, 16, 512).astype(
        jnp.bfloat16)
    w1 = wph0.astype(jnp.bfloat16)
    w2 = wph1.astype(jnp.bfloat16)
    w3 = _build_9tap(wph2)
    b3t = jnp.tile(b2, (1, 4))
    w4, b96 = _build_l4_banded(wph3, b3)

    b1sz, b2sz, b3sz = 32, 16, 16
    g1, g2, g3 = n // b1sz, n // b2sz, n // b3sz

    y1, s1, q1 = pl.pallas_call(
        functools.partial(_k1, bsz=b1sz),
        out_shape=[jax.ShapeDtypeStruct((n, 2, 2, 16, 256), jnp.bfloat16),
                   jax.ShapeDtypeStruct((g1, 1, 256), jnp.float32),
                   jax.ShapeDtypeStruct((g1, 1, 256), jnp.float32)],
        grid=(g1,),
        in_specs=[pl.BlockSpec((b1sz, 16, 512), lambda i: (i, 0, 0)),
                  pl.BlockSpec((4, 2048, 256), lambda i: (0, 0, 0)),
                  pl.BlockSpec((1, 256), lambda i: (0, 0))],
        out_specs=[pl.BlockSpec((b1sz, 2, 2, 16, 256),
                                lambda i: (i, 0, 0, 0, 0)),
                   pl.BlockSpec((1, 1, 256), lambda i: (i, 0, 0)),
                   pl.BlockSpec((1, 1, 256), lambda i: (i, 0, 0))],
        compiler_params=pltpu.CompilerParams(
            dimension_semantics=("parallel",),
            vmem_limit_bytes=56 * 1024 * 1024),
    )(xh, w1, b0)
    sc1, sh1 = _bn_affine(s1, q1, gamma0, beta0, n * 64)

    y2, s2, q2 = pl.pallas_call(
        functools.partial(_k2, bsz=b2sz),
        out_shape=[jax.ShapeDtypeStruct((n, 2, 2, 2, 2, 16, 128),
                                        jnp.bfloat16),
                   jax.ShapeDtypeStruct((g2, 1, 128), jnp.float32),
                   jax.ShapeDtypeStruct((g2, 1, 128), jnp.float32)],
        grid=(g2,),
        in_specs=[pl.BlockSpec((b2sz, 2, 2, 16, 256),
                               lambda i: (i, 0, 0, 0, 0)),
                  pl.BlockSpec((1, 256), lambda i: (0, 0)),
                  pl.BlockSpec((1, 256), lambda i: (0, 0)),
                  pl.BlockSpec((4, 1024, 128), lambda i: (0, 0, 0)),
                  pl.BlockSpec((1, 128), lambda i: (0, 0))],
        out_specs=[pl.BlockSpec((b2sz, 2, 2, 2, 2, 16, 128),
                                lambda i: (i, 0, 0, 0, 0, 0, 0)),
                   pl.BlockSpec((1, 1, 128), lambda i: (i, 0, 0)),
                   pl.BlockSpec((1, 1, 128), lambda i: (i, 0, 0))],
        compiler_params=pltpu.CompilerParams(
            dimension_semantics=("parallel",),
            vmem_limit_bytes=56 * 1024 * 1024),
    )(y1, sc1, sh1, w2, b1)
    sc2, sh2 = _bn_affine(s2, q2, gamma1, beta1, n * 256)

    y2n = y2.reshape(n, 4, 4, 16, 128)
    o = pl.pallas_call(
        functools.partial(_k34, bsz=b3sz),
        out_shape=jax.ShapeDtypeStruct((n, 8, 16, 96), jnp.bfloat16),
        grid=(g3,),
        in_specs=[pl.BlockSpec((b3sz, 4, 4, 16, 128),
                               lambda i: (i, 0, 0, 0, 0)),
                  pl.BlockSpec((1, 128), lambda i: (0, 0)),
                  pl.BlockSpec((1, 128), lambda i: (0, 0)),
                  pl.BlockSpec((1152, 256), lambda i: (0, 0)),
                  pl.BlockSpec((1, 256), lambda i: (0, 0)),
                  pl.BlockSpec((1920, 96), lambda i: (0, 0)),
                  pl.BlockSpec((1, 96), lambda i: (0, 0))],
        out_specs=pl.BlockSpec((b3sz, 8, 16, 96),
                               lambda i: (i, 0, 0, 0)),
        compiler_params=pltpu.CompilerParams(
            dimension_semantics=("parallel",),
            vmem_limit_bytes=56 * 1024 * 1024),
    )(y2n, sc2, sh2, w3, b3t, w4, b96)

    r = o.reshape(n, 8, 4, 4, 2, 16, 3)
    return r.transpose(0, 6, 2, 1, 4, 3, 5).reshape(
        n, 3, 64, 64).astype(jnp.float32)


# speedup vs baseline: 1.0446x; 1.0058x over previous
import functools

import jax
import jax.numpy as jnp
from jax.experimental import pallas as pl
from jax.experimental.pallas import tpu as pltpu

_EPS = 1e-5

_D = {0: (0, -1), 1: (1, 0)}
_PHASES = ((0, 0), (0, 1), (1, 0), (1, 1))


def _qshift_h(t, d):
    z = jnp.zeros_like(t[..., 0:4, :])
    if d == 1:
        return jnp.concatenate([t[..., 4:, :], z], axis=-2)
    return jnp.concatenate([z, t[..., :12, :]], axis=-2)


def _qshift_w(t, d):
    z = jnp.zeros_like(t[..., 0:1, :])
    if d == 1:
        s = jnp.concatenate([t[..., 1:, :], z], axis=-2)
        bad = 3
    else:
        s = jnp.concatenate([z, t[..., :-1, :]], axis=-2)
        bad = 0
    q = jax.lax.broadcasted_iota(jnp.int32, (16, 1), 0) % 4
    return jnp.where(q == bad, jnp.zeros_like(s), s)


def _tap_h(y, d):
    if d == 0:
        return y
    p = y.shape[1]
    if d == 1:
        wrap = _qshift_h(y[:, 0:1], 1)
        if p == 1:
            return wrap
        return jnp.concatenate([y[:, 1:], wrap], axis=1)
    wrap = _qshift_h(y[:, p - 1:p], -1)
    if p == 1:
        return wrap
    return jnp.concatenate([wrap, y[:, :p - 1]], axis=1)


def _tap_w(y, d):
    if d == 0:
        return y
    p = y.shape[2]
    if d == 1:
        wrap = _qshift_w(y[:, :, 0:1], 1)
        if p == 1:
            return wrap
        return jnp.concatenate([y[:, :, 1:], wrap], axis=2)
    wrap = _qshift_w(y[:, :, p - 1:p], -1)
    if p == 1:
        return wrap
    return jnp.concatenate([wrap, y[:, :, :p - 1]], axis=2)


def _taps9(y):
    bsz, p, _, s, c = y.shape
    zt = {}
    for dh in (-1, 0, 1):
        th = _tap_h(y, dh)
        for dw in (-1, 0, 1):
            zt[(dh, dw)] = _tap_w(th, dw).reshape(bsz * p * p * s, c)
    return zt


def _deconv_phases(y, w_ref, b_ref):
    zt = _taps9(y)
    outs = []
    for ph, (a, b) in enumerate(_PHASES):
        xt = jnp.concatenate(
            [zt[(dh, dw)] for dh in _D[a] for dw in _D[b]], axis=1)
        outs.append(jnp.dot(xt, w_ref[ph],
                            preferred_element_type=jnp.float32) + b_ref[...])
    return outs


def _k1(x_ref, w_ref, b_ref, o_ref, s_ref, q_ref, *, bsz):
    y = x_ref[...].reshape(bsz, 1, 1, 16, 512)
    outs = _deconv_phases(y, w_ref, b_ref)
    ssum = jnp.zeros((256,), jnp.float32)
    ssq = jnp.zeros((256,), jnp.float32)
    for ph, (a, b) in enumerate(_PHASES):
        o = outs[ph]
        ssum = ssum + jnp.sum(o, axis=0)
        ssq = ssq + jnp.sum(o * o, axis=0)
        o_ref[:, a, b] = o.astype(jnp.bfloat16).reshape(bsz, 16, 256)
    s_ref[...] = ssum.reshape(1, 1, 256)
    q_ref[...] = ssq.reshape(1, 1, 256)


def _k2(x_ref, sc_ref, sf_ref, w_ref, b_ref, o_ref, s_ref, q_ref, *, bsz):
    x = x_ref[...].astype(jnp.float32)
    x = jnp.maximum(x * sc_ref[...] + sf_ref[...], 0.0).astype(jnp.bfloat16)
    outs = _deconv_phases(x, w_ref, b_ref)
    ssum = jnp.zeros((128,), jnp.float32)
    ssq = jnp.zeros((128,), jnp.float32)
    for ph, (a, b) in enumerate(_PHASES):
        o = outs[ph]
        ssum = ssum + jnp.sum(o, axis=0)
        ssq = ssq + jnp.sum(o * o, axis=0)
        o_ref[:, :, a, :, b] = o.astype(jnp.bfloat16).reshape(
            bsz, 2, 2, 16, 128)
    s_ref[...] = ssum.reshape(1, 1, 128)
    q_ref[...] = ssq.reshape(1, 1, 128)


def _k34(x_ref, sc_ref, sf_ref, w3_ref, b3_ref, w4_ref, b4_ref, o_ref, *,
         bsz):
    x = x_ref[...].astype(jnp.float32)
    x = jnp.maximum(x * sc_ref[...] + sf_ref[...], 0.0).astype(jnp.bfloat16)

    zt = _taps9(x)
    xt = jnp.concatenate(
        [zt[(dh, dw)] for dh in (-1, 0, 1) for dw in (-1, 0, 1)], axis=1)
    r = jnp.dot(xt, w3_ref[...],
                preferred_element_type=jnp.float32) + b3_ref[...]
    r = jnp.maximum(r, 0.0).astype(jnp.bfloat16)

    r6 = r.reshape(bsz, 4, 4, 16, 2, 2, 64)
    y3f = r6.transpose(0, 1, 4, 3, 2, 5, 6).reshape(bsz, 8, 16, 512)

    cols = []
    for dh in (-1, 0, 1):
        t = _tap_h(y3f, dh)
        left = _qshift_w(t[..., 7 * 64:], -1)
        right = _qshift_w(t[..., :64], 1)
        cols.append(jnp.concatenate([left, t, right], axis=-1))
    xtb = jnp.concatenate(cols, axis=-1).reshape(bsz * 128, 1920)
    y4 = jnp.dot(xtb, w4_ref[...],
                 preferred_element_type=jnp.float32) + b4_ref[...]
    o_ref[...] = y4.astype(jnp.bfloat16).reshape(bsz, 8, 16, 96)


def _bn_affine(s, q, gamma, beta, count):
    inv = 1.0 / float(count)
    mean = jnp.sum(s, axis=0).reshape(1, -1) * inv
    var = jnp.sum(q, axis=0).reshape(1, -1) * inv - mean * mean
    scale = gamma * jax.lax.rsqrt(var + _EPS)
    shift = beta - mean * scale
    return scale, shift


def _build_9tap(wph):
    cin = wph.shape[1] // 4
    cout = wph.shape[2]
    w9 = jnp.zeros((9, cin, 4 * cout), wph.dtype)
    for ph, (a, b) in enumerate(_PHASES):
        for j in range(4):
            dh = _D[a][j // 2]
            dw = _D[b][j % 2]
            k = (dh + 1) * 3 + (dw + 1)
            w9 = w9.at[k, :, ph * cout:(ph + 1) * cout].set(
                wph[ph, j * cin:(j + 1) * cin, :])
    return w9.reshape(9 * cin, 4 * cout).astype(jnp.bfloat16)


def _build_l4_banded(wph3, b3):
    w4 = jnp.zeros((3, 10, 64, 2, 16, 3), wph3.dtype)
    for ah in (0, 1):
        for aw in (0, 1):
            ph = ah * 2 + aw
            for jh, dh in enumerate(_D[ah]):
                for jw, dw in enumerate(_D[aw]):
                    j = jh * 2 + jw
                    blk = wph3[ph, j * 64:(j + 1) * 64, :]
                    for psi in range(8):
                        if dw == -1 and psi == 0:
                            psa = 0
                        elif dw == 1 and psi == 7:
                            psa = 9
                        else:
                            psa = psi + dw + 1
                        w4 = w4.at[dh + 1, psa, :, ah,
                                   2 * psi + aw, :].set(blk)
    b96 = jnp.tile(b3, (1, 32))
    return w4.reshape(1920, 96).astype(jnp.bfloat16), b96


def kernel(wph0, b0, wph1, b1, wph2, b2, wph3, b3,
           gamma0, beta0, gamma1, beta1, x):
    n = x.shape[0]
    xh = jnp.transpose(x, (0, 2, 3, 1)).reshape(n, 16, 512).astype(
        jnp.bfloat16)
    w1 = wph0.astype(jnp.bfloat16)
    w2 = wph1.astype(jnp.bfloat16)
    w3 = _build_9tap(wph2)
    b3t = jnp.tile(b2, (1, 4))
    w4, b96 = _build_l4_banded(wph3, b3)

    b1sz, b2sz, b3sz = 64, 32, 16
    g1, g2, g3 = n // b1sz, n // b2sz, n // b3sz

    y1, s1, q1 = pl.pallas_call(
        functools.partial(_k1, bsz=b1sz),
        out_shape=[jax.ShapeDtypeStruct((n, 2, 2, 16, 256), jnp.bfloat16),
                   jax.ShapeDtypeStruct((g1, 1, 256), jnp.float32),
                   jax.ShapeDtypeStruct((g1, 1, 256), jnp.float32)],
        grid=(g1,),
        in_specs=[pl.BlockSpec((b1sz, 16, 512), lambda i: (i, 0, 0)),
                  pl.BlockSpec((4, 2048, 256), lambda i: (0, 0, 0)),
                  pl.BlockSpec((1, 256), lambda i: (0, 0))],
        out_specs=[pl.BlockSpec((b1sz, 2, 2, 16, 256),
                                lambda i: (i, 0, 0, 0, 0)),
                   pl.BlockSpec((1, 1, 256), lambda i: (i, 0, 0)),
                   pl.BlockSpec((1, 1, 256), lambda i: (i, 0, 0))],
        compiler_params=pltpu.CompilerParams(
            dimension_semantics=("parallel",),
            vmem_limit_bytes=56 * 1024 * 1024),
    )(xh, w1, b0)
    sc1, sh1 = _bn_affine(s1, q1, gamma0, beta0, n * 64)

    y2, s2, q2 = pl.pallas_call(
        functools.partial(_k2, bsz=b2sz),
        out_shape=[jax.ShapeDtypeStruct((n, 2, 2, 2, 2, 16, 128),
                                        jnp.bfloat16),
                   jax.ShapeDtypeStruct((g2, 1, 128), jnp.float32),
                   jax.ShapeDtypeStruct((g2, 1, 128), jnp.float32)],
        grid=(g2,),
        in_specs=[pl.BlockSpec((b2sz, 2, 2, 16, 256),
                               lambda i: (i, 0, 0, 0, 0)),
                  pl.BlockSpec((1, 256), lambda i: (0, 0)),
                  pl.BlockSpec((1, 256), lambda i: (0, 0)),
                  pl.BlockSpec((4, 1024, 128), lambda i: (0, 0, 0)),
                  pl.BlockSpec((1, 128), lambda i: (0, 0))],
        out_specs=[pl.BlockSpec((b2sz, 2, 2, 2, 2, 16, 128),
                                lambda i: (i, 0, 0, 0, 0, 0, 0)),
                   pl.BlockSpec((1, 1, 128), lambda i: (i, 0, 0)),
                   pl.BlockSpec((1, 1, 128), lambda i: (i, 0, 0))],
        compiler_params=pltpu.CompilerParams(
            dimension_semantics=("parallel",),
            vmem_limit_bytes=56 * 1024 * 1024),
    )(y1, sc1, sh1, w2, b1)
    sc2, sh2 = _bn_affine(s2, q2, gamma1, beta1, n * 256)

    y2n = y2.reshape(n, 4, 4, 16, 128)
    o = pl.pallas_call(
        functools.partial(_k34, bsz=b3sz),
        out_shape=jax.ShapeDtypeStruct((n, 8, 16, 96), jnp.bfloat16),
        grid=(g3,),
        in_specs=[pl.BlockSpec((b3sz, 4, 4, 16, 128),
                               lambda i: (i, 0, 0, 0, 0)),
                  pl.BlockSpec((1, 128), lambda i: (0, 0)),
                  pl.BlockSpec((1, 128), lambda i: (0, 0)),
                  pl.BlockSpec((1152, 256), lambda i: (0, 0)),
                  pl.BlockSpec((1, 256), lambda i: (0, 0)),
                  pl.BlockSpec((1920, 96), lambda i: (0, 0)),
                  pl.BlockSpec((1, 96), lambda i: (0, 0))],
        out_specs=pl.BlockSpec((b3sz, 8, 16, 96),
                               lambda i: (i, 0, 0, 0)),
        compiler_params=pltpu.CompilerParams(
            dimension_semantics=("parallel",),
            vmem_limit_bytes=56 * 1024 * 1024),
    )(y2n, sc2, sh2, w3, b3t, w4, b96)

    r = o.reshape(n, 8, 4, 4, 2, 16, 3)
    return r.transpose(0, 6, 2, 1, 4, 3, 5).reshape(
        n, 3, 64, 64).astype(jnp.float32)


# speedup vs baseline: 1.0849x; 1.0386x over previous
import functools

import jax
import jax.numpy as jnp
from jax.experimental import pallas as pl
from jax.experimental.pallas import tpu as pltpu

_EPS = 1e-5

_D = {0: (0, -1), 1: (1, 0)}
_PHASES = ((0, 0), (0, 1), (1, 0), (1, 1))


def _qshift_h(t, d):
    z = jnp.zeros_like(t[..., 0:4, :])
    if d == 1:
        return jnp.concatenate([t[..., 4:, :], z], axis=-2)
    return jnp.concatenate([z, t[..., :12, :]], axis=-2)


def _qshift_w(t, d):
    z = jnp.zeros_like(t[..., 0:1, :])
    if d == 1:
        s = jnp.concatenate([t[..., 1:, :], z], axis=-2)
        bad = 3
    else:
        s = jnp.concatenate([z, t[..., :-1, :]], axis=-2)
        bad = 0
    q = jax.lax.broadcasted_iota(jnp.int32, (16, 1), 0) % 4
    return jnp.where(q == bad, jnp.zeros_like(s), s)


def _tap_h(y, d):
    if d == 0:
        return y
    p = y.shape[1]
    if d == 1:
        wrap = _qshift_h(y[:, 0:1], 1)
        if p == 1:
            return wrap
        return jnp.concatenate([y[:, 1:], wrap], axis=1)
    wrap = _qshift_h(y[:, p - 1:p], -1)
    if p == 1:
        return wrap
    return jnp.concatenate([wrap, y[:, :p - 1]], axis=1)


def _tap_w(y, d):
    if d == 0:
        return y
    p = y.shape[2]
    if d == 1:
        wrap = _qshift_w(y[:, :, 0:1], 1)
        if p == 1:
            return wrap
        return jnp.concatenate([y[:, :, 1:], wrap], axis=2)
    wrap = _qshift_w(y[:, :, p - 1:p], -1)
    if p == 1:
        return wrap
    return jnp.concatenate([wrap, y[:, :, :p - 1]], axis=2)


def _taps9(y):
    bsz, p, _, s, c = y.shape
    zt = {}
    for dh in (-1, 0, 1):
        th = _tap_h(y, dh)
        for dw in (-1, 0, 1):
            zt[(dh, dw)] = _tap_w(th, dw).reshape(bsz * p * p * s, c)
    return zt


def _deconv_phases(y, w_ref, b_ref):
    zt = _taps9(y)
    outs = []
    for ph, (a, b) in enumerate(_PHASES):
        xt = jnp.concatenate(
            [zt[(dh, dw)] for dh in _D[a] for dw in _D[b]], axis=1)
        outs.append(jnp.dot(xt, w_ref[ph],
                            preferred_element_type=jnp.float32) + b_ref[...])
    return outs


def _k1(x_ref, w_ref, b_ref, o_ref, s_ref, q_ref, *, bsz):
    y = x_ref[...].reshape(bsz, 1, 1, 16, 512)
    outs = _deconv_phases(y, w_ref, b_ref)
    ssum = jnp.zeros((256,), jnp.float32)
    ssq = jnp.zeros((256,), jnp.float32)
    for ph, (a, b) in enumerate(_PHASES):
        o = outs[ph]
        ssum = ssum + jnp.sum(o, axis=0)
        ssq = ssq + jnp.sum(o * o, axis=0)
        o_ref[:, a, b] = o.astype(jnp.bfloat16).reshape(bsz, 16, 256)
    s_ref[...] = ssum.reshape(1, 1, 256)
    q_ref[...] = ssq.reshape(1, 1, 256)


def _k2(x_ref, sc_ref, sf_ref, w_ref, b_ref, o_ref, s_ref, q_ref, *, bsz):
    x = x_ref[...].astype(jnp.float32)
    x = jnp.maximum(x * sc_ref[...] + sf_ref[...], 0.0).astype(jnp.bfloat16)
    outs = _deconv_phases(x, w_ref, b_ref)
    ssum = jnp.zeros((128,), jnp.float32)
    ssq = jnp.zeros((128,), jnp.float32)
    for ph, (a, b) in enumerate(_PHASES):
        o = outs[ph]
        ssum = ssum + jnp.sum(o, axis=0)
        ssq = ssq + jnp.sum(o * o, axis=0)
        o_ref[:, :, a, :, b] = o.astype(jnp.bfloat16).reshape(
            bsz, 2, 2, 16, 128)
    s_ref[...] = ssum.reshape(1, 1, 128)
    q_ref[...] = ssq.reshape(1, 1, 128)


def _k34(x_ref, sc_ref, sf_ref, w3_ref, b3_ref, w4_ref, b4_ref, o_ref, *,
         bsz):
    x = x_ref[...].astype(jnp.float32)
    x = jnp.maximum(x * sc_ref[...] + sf_ref[...], 0.0).astype(jnp.bfloat16)

    cols3 = []
    for dh in (-1, 0, 1):
        t = _tap_h(x, dh)
        left = _qshift_w(t[..., 3 * 128:], -1)
        right = _qshift_w(t[..., :128], 1)
        cols3.append(jnp.concatenate([left, t, right], axis=-1))
    xt3 = jnp.concatenate(cols3, axis=-1).reshape(bsz * 64, 2304)
    r = jnp.dot(xt3, w3_ref[...],
                preferred_element_type=jnp.float32) + b3_ref[...]
    r = jnp.maximum(r, 0.0).astype(jnp.bfloat16)

    h0 = r[:, :512].reshape(bsz, 4, 16, 512)
    h1 = r[:, 512:].reshape(bsz, 4, 16, 512)
    y3f = jnp.stack([h0, h1], axis=2).reshape(bsz, 8, 16, 512)

    cols = []
    for dh in (-1, 0, 1):
        t = _tap_h(y3f, dh)
        left = _qshift_w(t[..., 7 * 64:], -1)
        right = _qshift_w(t[..., :64], 1)
        cols.append(jnp.concatenate([left, t, right], axis=-1))
    xtb = jnp.concatenate(cols, axis=-1).reshape(bsz * 128, 1920)
    y4 = jnp.dot(xtb, w4_ref[...],
                 preferred_element_type=jnp.float32) + b4_ref[...]
    o_ref[...] = y4.astype(jnp.bfloat16).reshape(bsz, 8, 16, 96)


def _bn_affine(s, q, gamma, beta, count):
    inv = 1.0 / float(count)
    mean = jnp.sum(s, axis=0).reshape(1, -1) * inv
    var = jnp.sum(q, axis=0).reshape(1, -1) * inv - mean * mean
    scale = gamma * jax.lax.rsqrt(var + _EPS)
    shift = beta - mean * scale
    return scale, shift


def _build_9tap(wph):
    cin = wph.shape[1] // 4
    cout = wph.shape[2]
    w9 = jnp.zeros((9, cin, 4 * cout), wph.dtype)
    for ph, (a, b) in enumerate(_PHASES):
        for j in range(4):
            dh = _D[a][j // 2]
            dw = _D[b][j % 2]
            k = (dh + 1) * 3 + (dw + 1)
            w9 = w9.at[k, :, ph * cout:(ph + 1) * cout].set(
                wph[ph, j * cin:(j + 1) * cin, :])
    return w9.reshape(9 * cin, 4 * cout).astype(jnp.bfloat16)


def _build_l3_banded(wph2, b2):
    w3 = jnp.zeros((3, 6, 128, 2, 8, 64), wph2.dtype)
    for ah in (0, 1):
        for aw in (0, 1):
            ph = ah * 2 + aw
            for jh, dh in enumerate(_D[ah]):
                for jw, dw in enumerate(_D[aw]):
                    j = jh * 2 + jw
                    blk = wph2[ph, j * 128:(j + 1) * 128, :]
                    for psi in range(4):
                        if dw == -1 and psi == 0:
                            psa = 0
                        elif dw == 1 and psi == 3:
                            psa = 5
                        else:
                            psa = psi + dw + 1
                        w3 = w3.at[dh + 1, psa, :, ah,
                                   2 * psi + aw, :].set(blk)
    b1024 = jnp.tile(b2, (1, 16))
    return w3.reshape(2304, 1024).astype(jnp.bfloat16), b1024


def _build_l4_banded(wph3, b3):
    w4 = jnp.zeros((3, 10, 64, 2, 16, 3), wph3.dtype)
    for ah in (0, 1):
        for aw in (0, 1):
            ph = ah * 2 + aw
            for jh, dh in enumerate(_D[ah]):
                for jw, dw in enumerate(_D[aw]):
                    j = jh * 2 + jw
                    blk = wph3[ph, j * 64:(j + 1) * 64, :]
                    for psi in range(8):
                        if dw == -1 and psi == 0:
                            psa = 0
                        elif dw == 1 and psi == 7:
                            psa = 9
                        else:
                            psa = psi + dw + 1
                        w4 = w4.at[dh + 1, psa, :, ah,
                                   2 * psi + aw, :].set(blk)
    b96 = jnp.tile(b3, (1, 32))
    return w4.reshape(1920, 96).astype(jnp.bfloat16), b96


def kernel(wph0, b0, wph1, b1, wph2, b2, wph3, b3,
           gamma0, beta0, gamma1, beta1, x):
    n = x.shape[0]
    xh = jnp.transpose(x, (0, 2, 3, 1)).reshape(n, 16, 512).astype(
        jnp.bfloat16)
    w1 = wph0.astype(jnp.bfloat16)
    w2 = wph1.astype(jnp.bfloat16)
    w3, b3t = _build_l3_banded(wph2, b2)
    w4, b96 = _build_l4_banded(wph3, b3)

    b1sz, b2sz, b3sz = 64, 32, 16
    g1, g2, g3 = n // b1sz, n // b2sz, n // b3sz

    y1, s1, q1 = pl.pallas_call(
        functools.partial(_k1, bsz=b1sz),
        out_shape=[jax.ShapeDtypeStruct((n, 2, 2, 16, 256), jnp.bfloat16),
                   jax.ShapeDtypeStruct((g1, 1, 256), jnp.float32),
                   jax.ShapeDtypeStruct((g1, 1, 256), jnp.float32)],
        grid=(g1,),
        in_specs=[pl.BlockSpec((b1sz, 16, 512), lambda i: (i, 0, 0)),
                  pl.BlockSpec((4, 2048, 256), lambda i: (0, 0, 0)),
                  pl.BlockSpec((1, 256), lambda i: (0, 0))],
        out_specs=[pl.BlockSpec((b1sz, 2, 2, 16, 256),
                                lambda i: (i, 0, 0, 0, 0)),
                   pl.BlockSpec((1, 1, 256), lambda i: (i, 0, 0)),
                   pl.BlockSpec((1, 1, 256), lambda i: (i, 0, 0))],
        compiler_params=pltpu.CompilerParams(
            dimension_semantics=("parallel",),
            vmem_limit_bytes=56 * 1024 * 1024),
    )(xh, w1, b0)
    sc1, sh1 = _bn_affine(s1, q1, gamma0, beta0, n * 64)

    y2, s2, q2 = pl.pallas_call(
        functools.partial(_k2, bsz=b2sz),
        out_shape=[jax.ShapeDtypeStruct((n, 2, 2, 2, 2, 16, 128),
                                        jnp.bfloat16),
                   jax.ShapeDtypeStruct((g2, 1, 128), jnp.float32),
                   jax.ShapeDtypeStruct((g2, 1, 128), jnp.float32)],
        grid=(g2,),
        in_specs=[pl.BlockSpec((b2sz, 2, 2, 16, 256),
                               lambda i: (i, 0, 0, 0, 0)),
                  pl.BlockSpec((1, 256), lambda i: (0, 0)),
                  pl.BlockSpec((1, 256), lambda i: (0, 0)),
                  pl.BlockSpec((4, 1024, 128), lambda i: (0, 0, 0)),
                  pl.BlockSpec((1, 128), lambda i: (0, 0))],
        out_specs=[pl.BlockSpec((b2sz, 2, 2, 2, 2, 16, 128),
                                lambda i: (i, 0, 0, 0, 0, 0, 0)),
                   pl.BlockSpec((1, 1, 128), lambda i: (i, 0, 0)),
                   pl.BlockSpec((1, 1, 128), lambda i: (i, 0, 0))],
        compiler_params=pltpu.CompilerParams(
            dimension_semantics=("parallel",),
            vmem_limit_bytes=56 * 1024 * 1024),
    )(y1, sc1, sh1, w2, b1)
    sc2, sh2 = _bn_affine(s2, q2, gamma1, beta1, n * 256)

    y2n = (y2.reshape(n, 4, 4, 16, 128).transpose(0, 1, 3, 2, 4)
           .reshape(n, 4, 16, 512))
    sc2f = jnp.tile(sc2, (1, 4))
    sh2f = jnp.tile(sh2, (1, 4))
    o = pl.pallas_call(
        functools.partial(_k34, bsz=b3sz),
        out_shape=jax.ShapeDtypeStruct((n, 8, 16, 96), jnp.bfloat16),
        grid=(g3,),
        in_specs=[pl.BlockSpec((b3sz, 4, 16, 512),
                               lambda i: (i, 0, 0, 0)),
                  pl.BlockSpec((1, 512), lambda i: (0, 0)),
                  pl.BlockSpec((1, 512), lambda i: (0, 0)),
                  pl.BlockSpec((2304, 1024), lambda i: (0, 0)),
                  pl.BlockSpec((1, 1024), lambda i: (0, 0)),
                  pl.BlockSpec((1920, 96), lambda i: (0, 0)),
                  pl.BlockSpec((1, 96), lambda i: (0, 0))],
        out_specs=pl.BlockSpec((b3sz, 8, 16, 96),
                               lambda i: (i, 0, 0, 0)),
        compiler_params=pltpu.CompilerParams(
            dimension_semantics=("parallel",),
            vmem_limit_bytes=56 * 1024 * 1024),
    )(y2n, sc2f, sh2f, w3, b3t, w4, b96)

    r = o.reshape(n, 8, 4, 4, 2, 16, 3)
    return r.transpose(0, 6, 2, 1, 4, 3, 5).reshape(
        n, 3, 64, 64).astype(jnp.float32)
